# baseline (speedup 1.0000x reference)
import functools

import jax
import jax.numpy as jnp
from jax import lax
from jax.experimental import pallas as pl
from jax.experimental.pallas import tpu as pltpu
from jax.experimental.pallas import tpu_sc as plsc

_NUM_CORES = 2
_NUM_SUBCORES = 16
_NUM_WORKERS = _NUM_CORES * _NUM_SUBCORES
_LANES = 16
_HI_MASK = -65536
_BLK = 2 * _LANES


def _rne16(x):
    return lax.shift_right_logical(
        x + 0x7FFF + (lax.shift_right_logical(x, 16) & 1), 16)


@functools.lru_cache(maxsize=None)
def _make_gather_sum(L, B, V):
    assert L % 8 == 0 and B == 1024 and V % _BLK == 0
    tile_rows = L // 8
    b_per_w = B // _NUM_WORKERS
    groups = b_per_w // _LANES
    n_blocks = V // _BLK
    n_rounds = 4
    blk_rnd = 50
    blk_round_total = blk_rnd * _NUM_SUBCORES
    f32_rnd = blk_rnd * _BLK
    words_rnd = blk_rnd * _LANES
    words_round_total = blk_round_total * _LANES
    stage_w = n_rounds * words_round_total
    assert (n_rounds - 1) * blk_round_total <= n_blocks \
        <= n_rounds * blk_round_total
    mesh = plsc.VectorSubcoreMesh(core_axis_name="c", subcore_axis_name="s")

    @functools.partial(
        pl.kernel,
        mesh=mesh,
        out_type=(
            jax.ShapeDtypeStruct((1, B), jnp.float32),
            jax.ShapeDtypeStruct((_NUM_CORES, stage_w), jnp.int32),
        ),
        scratch_types=[
            pltpu.VMEM((V // 2,), jnp.int32),
            pltpu.VMEM((tile_rows, 8, b_per_w), jnp.int32),
            pltpu.VMEM((f32_rnd,), jnp.float32),
            pltpu.VMEM((f32_rnd,), jnp.float32),
            pltpu.VMEM((words_rnd,), jnp.int32),
            pltpu.VMEM((b_per_w,), jnp.float32),
            pltpu.VMEM((_LANES,), jnp.float32),
            pltpu.SemaphoreType.DMA,
            pltpu.SemaphoreType.DMA,
            pltpu.SemaphoreType.DMA,
            pltpu.SemaphoreType.DMA,
        ],
        compiler_params=pltpu.CompilerParams(
            needs_layout_passes=False, use_tc_tiling_on_sc=False),
    )
    def gather_sum(w_hbm, tex_hbm, bias_hbm, out_hbm, stage_hbm, w_v, tex_v,
                   pin_0, pin_1, pout_v, out_v, bias_v, sem_w, sem_t,
                   sem_0, sem_1):
        core = lax.axis_index("c")
        sid = lax.axis_index("s")
        wid = sid * _NUM_CORES + core
        tile_col = wid // 4
        c0 = (wid % 4) * b_per_w
        cp_t = pltpu.async_copy(
            tex_hbm.at[:, tile_col, :, pl.ds(c0, b_per_w)], tex_v, sem_t)

        pins = (pin_0, pin_1)
        sems = (sem_0, sem_1)

        def issue_in(r):
            blk_r0 = r * blk_round_total + sid * blk_rnd
            f_want = blk_r0 * _BLK
            f_start = jnp.minimum(f_want, V - f32_rnd)
            cp = pltpu.async_copy(w_hbm.at[0, pl.ds(f_start, f32_rnd)],
                                  pins[r % 2], sems[r % 2])
            return cp, f_want - f_start, blk_r0

        inflight = [issue_in(0), issue_in(1)]
        pltpu.sync_copy(bias_hbm, bias_v.at[pl.ds(0, 1)])

        def pack_round(pin_v, local_off, blk0):
            def pack_body(i, carry):
                @pl.when(blk0 + i < n_blocks)
                def _():
                    base = local_off + i * _BLK
                    b0 = plsc.bitcast(pin_v[pl.ds(base, _LANES)], jnp.int32)
                    b1 = plsc.bitcast(pin_v[pl.ds(base + _LANES, _LANES)],
                                      jnp.int32)
                    pout_v[pl.ds(i * _LANES, _LANES)] = (
                        _rne16(b0) | (_rne16(b1) << 16))
                return carry

            lax.fori_loop(0, blk_rnd, pack_body, 0)

        chunk_cps = []
        for r in range(n_rounds):
            cp_in, local_off, blk_r0 = inflight[r]
            cp_in.wait()
            pack_round(pins[r % 2], local_off, blk_r0)
            if r + 2 < n_rounds:
                inflight.append(issue_in(r + 2))
            pltpu.sync_copy(
                pout_v,
                stage_hbm.at[core, pl.ds(blk_r0 * _LANES, words_rnd)])
            plsc.subcore_barrier()
            w0 = r * words_round_total
            wlen = min(words_round_total, V // 2 - w0)
            chunk_cps.append(pltpu.async_copy(
                stage_hbm.at[core, pl.ds(w0, wlen)],
                w_v.at[pl.ds(w0, wlen)], sem_w))

        cp_t.wait()
        for cp in chunk_cps:
            cp.wait()
        bias = bias_v[...][0]

        def body(tr, accs):
            out = list(accs)
            for sub in range(8):
                for g in range(groups):
                    idx = tex_v[tr, sub, pl.ds(g * _LANES, _LANES)]
                    gidx = ((idx >> 5) << 4) | (idx & 15)
                    word = plsc.load_gather(w_v, [gidx])
                    bits = jnp.where((idx & 16) == 16, word & _HI_MASK,
                                     word << 16)
                    out[g] = out[g] + plsc.bitcast(bits, jnp.float32)
            return tuple(out)

        init = tuple(jnp.zeros((_LANES,), jnp.float32) for _ in range(groups))
        accs = lax.fori_loop(0, tile_rows, body, init)
        for g in range(groups):
            out_v[pl.ds(g * _LANES, _LANES)] = accs[g] + bias
        pltpu.sync_copy(
            out_v, out_hbm.at[0, pl.ds(tile_col * 128 + c0, b_per_w)])

    return gather_sum


def kernel(text, W, b):
    L, B = text.shape
    V = W.shape[1]
    tex4 = text.reshape(L // 8, 8, B // 128, 128).transpose(0, 2, 1, 3)
    out, _ = _make_gather_sum(L, B, V)(W, tex4, b)
    return out.reshape(B, 1)


# speedup vs baseline: 1.0049x; 1.0049x over previous
import functools

import jax
import jax.numpy as jnp
from jax import lax
from jax.experimental import pallas as pl
from jax.experimental.pallas import tpu as pltpu
from jax.experimental.pallas import tpu_sc as plsc

_NUM_CORES = 2
_NUM_SUBCORES = 16
_NUM_WORKERS = _NUM_CORES * _NUM_SUBCORES
_LANES = 16
_HI_MASK = -65536
_BLK = 2 * _LANES


def _rne16(x):
    return lax.shift_right_logical(
        x + 0x7FFF + (lax.shift_right_logical(x, 16) & 1), 16)


@functools.lru_cache(maxsize=None)
def _make_gather_sum(L, B, V):
    assert L % 8 == 0 and B == 1024 and V % _BLK == 0
    tile_rows = L // 8
    b_per_w = B // _NUM_WORKERS
    groups = b_per_w // _LANES
    n_blocks = V // _BLK
    blk_rnd = 100
    blk_a = blk_rnd * _NUM_SUBCORES
    f32_rnd = blk_rnd * _BLK
    words_rnd = blk_rnd * _LANES
    stage_w = 2 * _NUM_SUBCORES * words_rnd
    words_a = blk_a * _LANES
    assert blk_a <= n_blocks <= 2 * blk_a
    mesh = plsc.VectorSubcoreMesh(core_axis_name="c", subcore_axis_name="s")

    @functools.partial(
        pl.kernel,
        mesh=mesh,
        out_type=(
            jax.ShapeDtypeStruct((1, B), jnp.float32),
            jax.ShapeDtypeStruct((_NUM_CORES, stage_w), jnp.int32),
        ),
        scratch_types=[
            pltpu.VMEM((V // 2,), jnp.int32),
            pltpu.VMEM((tile_rows, 8, b_per_w), jnp.int32),
            pltpu.VMEM((f32_rnd,), jnp.float32),
            pltpu.VMEM((f32_rnd,), jnp.float32),
            pltpu.VMEM((words_rnd,), jnp.int32),
            pltpu.VMEM((b_per_w,), jnp.float32),
            pltpu.VMEM((_LANES,), jnp.float32),
            pltpu.SemaphoreType.DMA,
            pltpu.SemaphoreType.DMA,
            pltpu.SemaphoreType.DMA,
            pltpu.SemaphoreType.DMA,
            pltpu.SemaphoreType.DMA,
        ],
        compiler_params=pltpu.CompilerParams(
            needs_layout_passes=False, use_tc_tiling_on_sc=False),
    )
    def gather_sum(w_hbm, tex_hbm, bias_hbm, out_hbm, stage_hbm, w_v, tex_v,
                   pin_a, pin_b, pout_v, out_v, bias_v, sem_w, sem_w2,
                   sem_t, sem_a, sem_b):
        core = lax.axis_index("c")
        sid = lax.axis_index("s")
        wid = sid * _NUM_CORES + core
        tile_col = wid // 4
        c0 = (wid % 4) * b_per_w
        cp_t = pltpu.async_copy(
            tex_hbm.at[:, tile_col, :, pl.ds(c0, b_per_w)], tex_v, sem_t)

        blk_a0 = sid * blk_rnd
        blk_b0 = blk_a + sid * blk_rnd
        fa_start = blk_a0 * _BLK
        fb_want = blk_b0 * _BLK
        fb_start = jnp.minimum(fb_want, V - f32_rnd)
        fb_off = fb_want - fb_start
        cp_a = pltpu.async_copy(w_hbm.at[0, pl.ds(fa_start, f32_rnd)], pin_a,
                                sem_a)
        cp_b = pltpu.async_copy(w_hbm.at[0, pl.ds(fb_start, f32_rnd)], pin_b,
                                sem_b)
        pltpu.sync_copy(bias_hbm, bias_v.at[pl.ds(0, 1)])

        def pack_round(pin_v, local_off, blk0):
            def pack_body(i, carry):
                @pl.when(blk0 + i < n_blocks)
                def _():
                    base = local_off + i * _BLK
                    b0 = plsc.bitcast(pin_v[pl.ds(base, _LANES)], jnp.int32)
                    b1 = plsc.bitcast(pin_v[pl.ds(base + _LANES, _LANES)],
                                      jnp.int32)
                    pout_v[pl.ds(i * _LANES, _LANES)] = (
                        _rne16(b0) | (_rne16(b1) << 16))
                return carry

            lax.fori_loop(0, blk_rnd, pack_body, 0)

        cp_a.wait()
        pack_round(pin_a, 0, blk_a0)
        pltpu.sync_copy(pout_v,
                        stage_hbm.at[core, pl.ds(blk_a0 * _LANES, words_rnd)])
        plsc.subcore_barrier()
        cp_w = pltpu.async_copy(stage_hbm.at[core, pl.ds(0, words_a)],
                                w_v.at[pl.ds(0, words_a)], sem_w)
        cp_b.wait()
        pack_round(pin_b, fb_off, blk_b0)
        pltpu.sync_copy(pout_v,
                        stage_hbm.at[core, pl.ds(blk_b0 * _LANES, words_rnd)])
        plsc.subcore_barrier()
        cp_w2 = pltpu.async_copy(
            stage_hbm.at[core, pl.ds(words_a, V // 2 - words_a)],
            w_v.at[pl.ds(words_a, V // 2 - words_a)], sem_w2)

        cp_t.wait()
        cp_w.wait()
        cp_w2.wait()
        bias = bias_v[...][0]

        def body(tr, accs):
            out = list(accs)
            for sub in range(8):
                for g in range(groups):
                    idx = tex_v[tr, sub, pl.ds(g * _LANES, _LANES)]
                    gidx = ((idx >> 5) << 4) | (idx & 15)
                    word = plsc.load_gather(w_v, [gidx])
                    bits = jnp.where((idx & 16) == 16, word & _HI_MASK,
                                     word << 16)
                    out[g] = out[g] + plsc.bitcast(bits, jnp.float32)
            return tuple(out)

        init = tuple(jnp.zeros((_LANES,), jnp.float32) for _ in range(groups))
        accs = lax.fori_loop(0, tile_rows, body, init)
        for g in range(groups):
            out_v[pl.ds(g * _LANES, _LANES)] = accs[g] + bias
        pltpu.sync_copy(
            out_v, out_hbm.at[0, pl.ds(tile_col * 128 + c0, b_per_w)])

    return gather_sum


def kernel(text, W, b):
    L, B = text.shape
    V = W.shape[1]
    tex4 = text.reshape(L // 8, 8, B // 128, 128).transpose(0, 2, 1, 3)
    out, _ = _make_gather_sum(L, B, V)(W, tex4, b)
    return out.reshape(B, 1)
